# baseline (speedup 1.0000x reference)
import functools

import jax
import jax.numpy as jnp
from jax import lax
from jax.experimental import pallas as pl
from jax.experimental.pallas import tpu as pltpu
from jax.experimental.pallas import tpu_sc as plsc

NC = 2
NS = 16
TB = 256
FC = 128


EB = 125


@functools.cache
def _sc_agg(NP, E, nch, split=False):
    Fc = FC
    G = 8 if split else 16
    R = E // EB
    nblk = R // NS // (2 if split else 1)
    nsb = nblk // G
    rows_per_tile = NP // NS
    n_arr = 2 * nch if split else nch
    mesh = plsc.VectorSubcoreMesh(core_axis_name="c", subcore_axis_name="s",
                                  num_cores=NC, num_subcores=NS)

    @functools.partial(
        pl.kernel,
        mesh=mesh,
        out_type=[jax.ShapeDtypeStruct((NP, Fc), jnp.float32)
                  for _ in range(n_arr)],
        scratch_types=[
            pltpu.VMEM((2, 2, G, EB), jnp.int32),
            pltpu.VMEM((2, EB, Fc), jnp.float32),
            pltpu.VMEM_SHARED((NP, Fc), jnp.float32),
            pltpu.SemaphoreType.DMA,
            pltpu.SemaphoreType.DMA,
            pltpu.SemaphoreType.DMA,
            pltpu.SemaphoreType.DMA,
        ],
    )
    def agg(idx_hbm, *rest):
        x_chunks = rest[:n_arr]
        out_chunks = rest[n_arr:2 * n_arr]
        idxw, bufs, acc, semi0, semi1, semg0, semg1 = rest[2 * n_arr:]
        semi = (semi0, semi1)
        semg = (semg0, semg1)
        c = lax.axis_index("c")
        s = lax.axis_index("s")
        r0 = s * rows_per_tile
        tbase = (c * (R // 2) if split else 0) + s * nblk

        def istart(g, p):
            pltpu.async_copy(idx_hbm.at[0, pl.ds(tbase + g * G, G)],
                             idxw.at[p, 0], semi[p])
            pltpu.async_copy(idx_hbm.at[1, pl.ds(tbase + g * G, G)],
                             idxw.at[p, 1], semi[p])

        def iwait(p):
            pltpu.make_async_copy(idx_hbm.at[0, pl.ds(0, G)], idxw.at[p, 0],
                                  semi[p]).wait()
            pltpu.make_async_copy(idx_hbm.at[0, pl.ds(0, G)], idxw.at[p, 1],
                                  semi[p]).wait()

        for k in range(n_arr):
            @pl.when(c == (k % NC))
            def _(k=k):
                x_h = x_chunks[k]
                o_h = out_chunks[k]
                pltpu.sync_copy(x_h.at[pl.ds(r0, rows_per_tile)],
                                acc.at[pl.ds(r0, rows_per_tile)])
                plsc.subcore_barrier()

                def gstart(p, j, b):
                    pltpu.async_copy(x_h.at[idxw.at[p, 0, j]],
                                     bufs.at[b], semg[b])

                def gwait(b):
                    pltpu.make_async_copy(x_h.at[idxw.at[0, 0, 0]],
                                          bufs.at[b], semg[b]).wait()

                def sblock(g, p):
                    iwait(p)

                    @pl.when(g + 1 < nsb)
                    def _():
                        istart(g + 1, 1 - p)

                    gstart(p, 0, 0)
                    gstart(p, 1, 1)
                    for j in range(G):
                        b = j % 2
                        gwait(b)
                        pltpu.sync_copy(bufs.at[b],
                                        acc.at[idxw.at[p, 1, j]],
                                        add=True)
                        if j + 2 < G:
                            gstart(p, j + 2, b)

                istart(0, 0)

                def sbpair(t, carry):
                    sblock(2 * t, 0)
                    sblock(2 * t + 1, 1)
                    return carry

                lax.fori_loop(0, nsb // 2, sbpair, 0)
                if nsb % 2:
                    sblock(nsb - 1, 0)
                plsc.subcore_barrier()
                pltpu.sync_copy(acc.at[pl.ds(r0, rows_per_tile)],
                                o_h.at[pl.ds(r0, rows_per_tile)])
                plsc.subcore_barrier()

    return agg


@functools.cache
def _sc_deg(NP, E, eb):
    EB = eb
    R = E // EB
    nblk = R // NS // 2
    rows_per_tile = NP // NS
    mesh = plsc.VectorSubcoreMesh(core_axis_name="c", subcore_axis_name="s",
                                  num_cores=NC, num_subcores=NS)

    @functools.partial(
        pl.kernel,
        mesh=mesh,
        out_type=[jax.ShapeDtypeStruct((NP, 16), jnp.float32)
                  for _ in range(2)],
        scratch_types=[
            pltpu.VMEM((nblk, EB), jnp.int32),
            pltpu.VMEM((EB, 16), jnp.float32),
            pltpu.VMEM_SHARED((NP, 16), jnp.float32),
        ],
    )
    def deg_kernel(idx_hbm, ones_hbm, zeros_hbm, out0, out1,
                   dst_v, ones_v, acc):
        c = lax.axis_index("c")
        s = lax.axis_index("s")
        r0 = s * rows_per_tile
        pltpu.sync_copy(idx_hbm.at[1, pl.ds(c * (R // 2) + s * nblk, nblk)],
                        dst_v)

        def fill(j, carry):
            ones_v[j, :] = jnp.ones((16,), jnp.float32)
            return carry

        lax.fori_loop(0, EB, fill, 0)
        for k in range(2):
            @pl.when(c == k)
            def _(k=k):
                init_h = ones_hbm if k == 0 else zeros_hbm
                o_h = out0 if k == 0 else out1
                pltpu.sync_copy(init_h.at[pl.ds(r0, rows_per_tile)],
                                acc.at[pl.ds(r0, rows_per_tile)])
                plsc.subcore_barrier()

                def blk(j, carry):
                    pltpu.sync_copy(ones_v, acc.at[dst_v.at[j]], add=True)
                    return carry

                lax.fori_loop(0, nblk, blk, 0)
                plsc.subcore_barrier()
                pltpu.sync_copy(acc.at[pl.ds(r0, rows_per_tile)],
                                o_h.at[pl.ds(r0, rows_per_tile)])

    return deg_kernel


@functools.cache
def _t1_scale_split(NP, IN):
    nch = IN // FC

    def body(x_ref, deg0_ref, deg1_ref, dinv_ref, *outs):
        deg = deg0_ref[...][:, :1] + deg1_ref[...][:, :1]
        d = lax.rsqrt(deg)
        dinv_ref[...] = jnp.broadcast_to(d, (TB, 16))
        xs = x_ref[...] * d
        for k in range(nch):
            outs[k][...] = xs[:, k * FC:(k + 1) * FC]

    return pl.pallas_call(
        body,
        grid=(NP // TB,),
        in_specs=[pl.BlockSpec((TB, IN), lambda i: (i, 0)),
                  pl.BlockSpec((TB, 16), lambda i: (i, 0)),
                  pl.BlockSpec((TB, 16), lambda i: (i, 0))],
        out_specs=[pl.BlockSpec((TB, 16), lambda i: (i, 0))] +
                  [pl.BlockSpec((TB, FC), lambda i: (i, 0))] * nch,
        out_shape=[jax.ShapeDtypeStruct((NP, 16), jnp.float32)] +
                  [jax.ShapeDtypeStruct((NP, FC), jnp.float32)] * nch,
    )


@functools.cache
def _t2_double_matmul(NP, IN, H):
    nin = IN // FC
    nout = H // FC

    def body(*refs):
        ins = refs[:nin]
        dinv, w1t, b1, w2t = refs[nin:nin + 4]
        outs = refs[nin + 4:]
        d = dinv[...][:, :1]
        a = jnp.concatenate([r[...] for r in ins], axis=1) * d
        h1 = jnp.dot(a.astype(jnp.bfloat16), w1t[...],
                     preferred_element_type=jnp.float32)
        h1 = jnp.maximum(h1 + b1[...], 0.0)
        g = jnp.dot(h1.astype(jnp.bfloat16), w2t[...],
                    preferred_element_type=jnp.float32) * d
        for k in range(nout):
            outs[k][...] = g[:, k * FC:(k + 1) * FC]

    return pl.pallas_call(
        body,
        grid=(NP // TB,),
        in_specs=[pl.BlockSpec((TB, FC), lambda i: (i, 0))] * nin + [
            pl.BlockSpec((TB, 16), lambda i: (i, 0)),
            pl.BlockSpec((IN, H), lambda i: (0, 0)),
            pl.BlockSpec((1, H), lambda i: (0, 0)),
            pl.BlockSpec((H, H), lambda i: (0, 0)),
        ],
        out_specs=[pl.BlockSpec((TB, FC), lambda i: (i, 0))] * nout,
        out_shape=[jax.ShapeDtypeStruct((NP, FC), jnp.float32)] * nout,
    )


@functools.cache
def _t3_out_matmul(NP, N, H, CP):
    nin = H // FC

    def body(*refs):
        ins = refs[:nin]
        dinv, b2, w3t, h2_ref, g2_ref = refs[nin:]
        d = dinv[...][:, :1]
        agg = jnp.concatenate([r[...] for r in ins], axis=1)
        h2 = jnp.maximum(agg * d + b2[...], 0.0)
        h2_ref[...] = h2
        g2_ref[...] = jnp.dot(h2.astype(jnp.bfloat16), w3t[...],
                              preferred_element_type=jnp.float32) * d

    return pl.pallas_call(
        body,
        grid=(NP // TB,),
        in_specs=[pl.BlockSpec((TB, FC), lambda i: (i, 0))] * nin + [
            pl.BlockSpec((TB, 16), lambda i: (i, 0)),
            pl.BlockSpec((1, H), lambda i: (0, 0)),
            pl.BlockSpec((H, CP), lambda i: (0, 0)),
        ],
        out_specs=[pl.BlockSpec((TB, H), lambda i: (i, 0)),
                   pl.BlockSpec((TB, CP), lambda i: (i, 0))],
        out_shape=[jax.ShapeDtypeStruct((N, H), jnp.float32),
                   jax.ShapeDtypeStruct((NP, CP), jnp.float32)],
    )


@functools.cache
def _t4_final(NP, N, C, CP):
    def body(p0_ref, p1_ref, dinv_ref, b3_ref, o_ref):
        v = (p0_ref[...] + p1_ref[...]) * dinv_ref[...][:, :1]
        o_ref[...] = v[:, :C] + b3_ref[...]

    return pl.pallas_call(
        body,
        grid=(NP // TB,),
        in_specs=[pl.BlockSpec((TB, CP), lambda i: (i, 0)),
                  pl.BlockSpec((TB, CP), lambda i: (i, 0)),
                  pl.BlockSpec((TB, 16), lambda i: (i, 0)),
                  pl.BlockSpec((1, C), lambda i: (0, 0))],
        out_specs=pl.BlockSpec((TB, C), lambda i: (i, 0)),
        out_shape=jax.ShapeDtypeStruct((N, C), jnp.float32),
    )


def kernel(x, edge_index, W1, b1, W2, b2, W3, b3):
    N, IN = x.shape
    E = edge_index.shape[1]
    H = W1.shape[0]
    C = W3.shape[0]
    NP = -(-N // TB) * TB
    CP = 128

    idx3 = edge_index.reshape(2, E // EB, EB)

    ones16 = jnp.ones((NP, 16), jnp.float32)
    zeros16 = jnp.zeros((NP, 16), jnp.float32)
    zerosCP = jnp.zeros((NP, CP), jnp.float32)

    deg0, deg1 = _sc_deg(NP, E, EB)(idx3, ones16, zeros16)

    t1 = _t1_scale_split(NP, IN)
    dinv16, *xc = t1(x, deg0, deg1)

    a = _sc_agg(NP, E, IN // FC)(idx3, *xc)

    t2 = _t2_double_matmul(NP, IN, H)
    g1 = t2(*a, dinv16, W1.T.astype(jnp.bfloat16), b1[None],
            W2.T.astype(jnp.bfloat16))

    m = _sc_agg(NP, E, H // FC)(idx3, *g1)

    t3 = _t3_out_matmul(NP, N, H, CP)
    w3tp = jnp.pad(W3.T, ((0, 0), (0, CP - C))).astype(jnp.bfloat16)
    h2p, g2 = t3(*m, dinv16, b2[None], w3tp)

    p0, p1 = _sc_agg(NP, E, 1, True)(idx3, g2, zerosCP)

    t4 = _t4_final(NP, N, C, CP)
    outp = t4(p0, p1, dinv16, b3[None])

    return h2p, outp


# speedup vs baseline: 1.1207x; 1.1207x over previous
import functools

import jax
import jax.numpy as jnp
from jax import lax
from jax.experimental import pallas as pl
from jax.experimental.pallas import tpu as pltpu
from jax.experimental.pallas import tpu_sc as plsc

NC = 2
NS = 16
TB = 1024
FC = 128


EB = 125


@functools.cache
def _sc_agg(NP, E, nch, split=False):
    Fc = FC
    G = 8 if split else 16
    R = E // EB
    nblk = R // NS // (2 if split else 1)
    nsb = nblk // G
    rows_per_tile = NP // NS
    n_arr = 2 * nch if split else nch
    mesh = plsc.VectorSubcoreMesh(core_axis_name="c", subcore_axis_name="s",
                                  num_cores=NC, num_subcores=NS)

    @functools.partial(
        pl.kernel,
        mesh=mesh,
        out_type=[jax.ShapeDtypeStruct((NP, Fc), jnp.float32)
                  for _ in range(n_arr)],
        scratch_types=[
            pltpu.VMEM((2, 2, G, EB), jnp.int32),
            pltpu.VMEM((2, EB, Fc), jnp.float32),
            pltpu.VMEM_SHARED((NP, Fc), jnp.float32),
            pltpu.SemaphoreType.DMA,
            pltpu.SemaphoreType.DMA,
            pltpu.SemaphoreType.DMA,
            pltpu.SemaphoreType.DMA,
        ],
    )
    def agg(idx_hbm, *rest):
        x_chunks = rest[:n_arr]
        out_chunks = rest[n_arr:2 * n_arr]
        idxw, bufs, acc, semi0, semi1, semg0, semg1 = rest[2 * n_arr:]
        semi = (semi0, semi1)
        semg = (semg0, semg1)
        c = lax.axis_index("c")
        s = lax.axis_index("s")
        r0 = s * rows_per_tile
        tbase = (c * (R // 2) if split else 0) + s * nblk

        def istart(g, p):
            pltpu.async_copy(idx_hbm.at[0, pl.ds(tbase + g * G, G)],
                             idxw.at[p, 0], semi[p])
            pltpu.async_copy(idx_hbm.at[1, pl.ds(tbase + g * G, G)],
                             idxw.at[p, 1], semi[p])

        def iwait(p):
            pltpu.make_async_copy(idx_hbm.at[0, pl.ds(0, G)], idxw.at[p, 0],
                                  semi[p]).wait()
            pltpu.make_async_copy(idx_hbm.at[0, pl.ds(0, G)], idxw.at[p, 1],
                                  semi[p]).wait()

        for k in range(n_arr):
            @pl.when(c == (k % NC))
            def _(k=k):
                x_h = x_chunks[k]
                o_h = out_chunks[k]
                pltpu.sync_copy(x_h.at[pl.ds(r0, rows_per_tile)],
                                acc.at[pl.ds(r0, rows_per_tile)])
                plsc.subcore_barrier()

                def gstart(p, j, b):
                    pltpu.async_copy(x_h.at[idxw.at[p, 0, j]],
                                     bufs.at[b], semg[b])

                def gwait(b):
                    pltpu.make_async_copy(x_h.at[idxw.at[0, 0, 0]],
                                          bufs.at[b], semg[b]).wait()

                def sblock(g, p):
                    iwait(p)

                    @pl.when(g + 1 < nsb)
                    def _():
                        istart(g + 1, 1 - p)

                    gstart(p, 0, 0)
                    gstart(p, 1, 1)
                    for j in range(G):
                        b = j % 2
                        gwait(b)
                        pltpu.sync_copy(bufs.at[b],
                                        acc.at[idxw.at[p, 1, j]],
                                        add=True)
                        if j + 2 < G:
                            gstart(p, j + 2, b)

                istart(0, 0)

                def sbpair(t, carry):
                    sblock(2 * t, 0)
                    sblock(2 * t + 1, 1)
                    return carry

                lax.fori_loop(0, nsb // 2, sbpair, 0)
                if nsb % 2:
                    sblock(nsb - 1, 0)
                plsc.subcore_barrier()
                pltpu.sync_copy(acc.at[pl.ds(r0, rows_per_tile)],
                                o_h.at[pl.ds(r0, rows_per_tile)])
                plsc.subcore_barrier()

    return agg


@functools.cache
def _sc_deg(NP, E, eb):
    EB = eb
    R = E // EB
    nblk = R // NS // 2
    rows_per_tile = NP // NS
    mesh = plsc.VectorSubcoreMesh(core_axis_name="c", subcore_axis_name="s",
                                  num_cores=NC, num_subcores=NS)

    @functools.partial(
        pl.kernel,
        mesh=mesh,
        out_type=[jax.ShapeDtypeStruct((NP, 16), jnp.float32)
                  for _ in range(2)],
        scratch_types=[
            pltpu.VMEM((nblk, EB), jnp.int32),
            pltpu.VMEM((EB, 16), jnp.float32),
            pltpu.VMEM_SHARED((NP, 16), jnp.float32),
        ],
    )
    def deg_kernel(idx_hbm, ones_hbm, zeros_hbm, out0, out1,
                   dst_v, ones_v, acc):
        c = lax.axis_index("c")
        s = lax.axis_index("s")
        r0 = s * rows_per_tile
        pltpu.sync_copy(idx_hbm.at[1, pl.ds(c * (R // 2) + s * nblk, nblk)],
                        dst_v)

        def fill(j, carry):
            ones_v[j, :] = jnp.ones((16,), jnp.float32)
            return carry

        lax.fori_loop(0, EB, fill, 0)
        for k in range(2):
            @pl.when(c == k)
            def _(k=k):
                init_h = ones_hbm if k == 0 else zeros_hbm
                o_h = out0 if k == 0 else out1
                pltpu.sync_copy(init_h.at[pl.ds(r0, rows_per_tile)],
                                acc.at[pl.ds(r0, rows_per_tile)])
                plsc.subcore_barrier()

                def blk(j, carry):
                    pltpu.sync_copy(ones_v, acc.at[dst_v.at[j]], add=True)
                    return carry

                lax.fori_loop(0, nblk, blk, 0)
                plsc.subcore_barrier()
                pltpu.sync_copy(acc.at[pl.ds(r0, rows_per_tile)],
                                o_h.at[pl.ds(r0, rows_per_tile)])

    return deg_kernel


@functools.cache
def _t1_scale_split(NP, IN):
    nch = IN // FC

    def body(x_ref, deg0_ref, deg1_ref, dinv_ref, *outs):
        deg = deg0_ref[...][:, :1] + deg1_ref[...][:, :1]
        d = lax.rsqrt(deg)
        dinv_ref[...] = jnp.broadcast_to(d, (TB, 16))
        xs = x_ref[...] * d
        for k in range(nch):
            outs[k][...] = xs[:, k * FC:(k + 1) * FC]

    return pl.pallas_call(
        body,
        grid=(NP // TB,),
        in_specs=[pl.BlockSpec((TB, IN), lambda i: (i, 0)),
                  pl.BlockSpec((TB, 16), lambda i: (i, 0)),
                  pl.BlockSpec((TB, 16), lambda i: (i, 0))],
        out_specs=[pl.BlockSpec((TB, 16), lambda i: (i, 0))] +
                  [pl.BlockSpec((TB, FC), lambda i: (i, 0))] * nch,
        out_shape=[jax.ShapeDtypeStruct((NP, 16), jnp.float32)] +
                  [jax.ShapeDtypeStruct((NP, FC), jnp.float32)] * nch,
    )


@functools.cache
def _t2_double_matmul(NP, IN, H):
    nin = IN // FC
    nout = H // FC

    def body(*refs):
        ins = refs[:nin]
        dinv, w1t, b1, w2t = refs[nin:nin + 4]
        outs = refs[nin + 4:]
        d = dinv[...][:, :1]
        a = jnp.concatenate([r[...] for r in ins], axis=1) * d
        h1 = jnp.dot(a.astype(jnp.bfloat16), w1t[...],
                     preferred_element_type=jnp.float32)
        h1 = jnp.maximum(h1 + b1[...], 0.0)
        g = jnp.dot(h1.astype(jnp.bfloat16), w2t[...],
                    preferred_element_type=jnp.float32) * d
        for k in range(nout):
            outs[k][...] = g[:, k * FC:(k + 1) * FC]

    return pl.pallas_call(
        body,
        grid=(NP // TB,),
        in_specs=[pl.BlockSpec((TB, FC), lambda i: (i, 0))] * nin + [
            pl.BlockSpec((TB, 16), lambda i: (i, 0)),
            pl.BlockSpec((IN, H), lambda i: (0, 0)),
            pl.BlockSpec((1, H), lambda i: (0, 0)),
            pl.BlockSpec((H, H), lambda i: (0, 0)),
        ],
        out_specs=[pl.BlockSpec((TB, FC), lambda i: (i, 0))] * nout,
        out_shape=[jax.ShapeDtypeStruct((NP, FC), jnp.float32)] * nout,
    )


@functools.cache
def _t3_out_matmul(NP, N, H, CP):
    nin = H // FC

    def body(*refs):
        ins = refs[:nin]
        dinv, b2, w3t, h2_ref, g2_ref = refs[nin:]
        d = dinv[...][:, :1]
        agg = jnp.concatenate([r[...] for r in ins], axis=1)
        h2 = jnp.maximum(agg * d + b2[...], 0.0)
        h2_ref[...] = h2
        g2_ref[...] = jnp.dot(h2.astype(jnp.bfloat16), w3t[...],
                              preferred_element_type=jnp.float32) * d

    return pl.pallas_call(
        body,
        grid=(NP // TB,),
        in_specs=[pl.BlockSpec((TB, FC), lambda i: (i, 0))] * nin + [
            pl.BlockSpec((TB, 16), lambda i: (i, 0)),
            pl.BlockSpec((1, H), lambda i: (0, 0)),
            pl.BlockSpec((H, CP), lambda i: (0, 0)),
        ],
        out_specs=[pl.BlockSpec((TB, H), lambda i: (i, 0)),
                   pl.BlockSpec((TB, CP), lambda i: (i, 0))],
        out_shape=[jax.ShapeDtypeStruct((N, H), jnp.float32),
                   jax.ShapeDtypeStruct((NP, CP), jnp.float32)],
    )


@functools.cache
def _t4_final(NP, N, C, CP):
    def body(p0_ref, p1_ref, dinv_ref, b3_ref, o_ref):
        v = (p0_ref[...] + p1_ref[...]) * dinv_ref[...][:, :1]
        o_ref[...] = v[:, :C] + b3_ref[...]

    return pl.pallas_call(
        body,
        grid=(NP // TB,),
        in_specs=[pl.BlockSpec((TB, CP), lambda i: (i, 0)),
                  pl.BlockSpec((TB, CP), lambda i: (i, 0)),
                  pl.BlockSpec((TB, 16), lambda i: (i, 0)),
                  pl.BlockSpec((1, C), lambda i: (0, 0))],
        out_specs=pl.BlockSpec((TB, C), lambda i: (i, 0)),
        out_shape=jax.ShapeDtypeStruct((N, C), jnp.float32),
    )


def kernel(x, edge_index, W1, b1, W2, b2, W3, b3):
    N, IN = x.shape
    E = edge_index.shape[1]
    H = W1.shape[0]
    C = W3.shape[0]
    NP = -(-N // TB) * TB
    CP = 128

    idx3 = edge_index.reshape(2, E // EB, EB)

    ones16 = jnp.ones((NP, 16), jnp.float32)
    zeros16 = jnp.zeros((NP, 16), jnp.float32)
    zerosCP = jnp.zeros((NP, CP), jnp.float32)

    deg0, deg1 = _sc_deg(NP, E, EB)(idx3, ones16, zeros16)

    t1 = _t1_scale_split(NP, IN)
    dinv16, *xc = t1(x, deg0, deg1)

    a = _sc_agg(NP, E, IN // FC)(idx3, *xc)

    t2 = _t2_double_matmul(NP, IN, H)
    g1 = t2(*a, dinv16, W1.T.astype(jnp.bfloat16), b1[None],
            W2.T.astype(jnp.bfloat16))

    m = _sc_agg(NP, E, H // FC)(idx3, *g1)

    t3 = _t3_out_matmul(NP, N, H, CP)
    w3tp = jnp.pad(W3.T, ((0, 0), (0, CP - C))).astype(jnp.bfloat16)
    h2p, g2 = t3(*m, dinv16, b2[None], w3tp)

    p0, p1 = _sc_agg(NP, E, 1, True)(idx3, g2, zerosCP)

    t4 = _t4_final(NP, N, C, CP)
    outp = t4(p0, p1, dinv16, b3[None])

    return h2p, outp


# speedup vs baseline: 1.1861x; 1.0584x over previous
import functools

import jax
import jax.numpy as jnp
from jax import lax
from jax.experimental import pallas as pl
from jax.experimental.pallas import tpu as pltpu
from jax.experimental.pallas import tpu_sc as plsc

NC = 2
NS = 16
TB = 1024
FC = 128


EB = 125


@functools.cache
def _sc_agg(NP, E, nch, split=False):
    Fc = FC
    G = 8 if split else 16
    R = E // EB
    nblk = R // NS // (2 if split else 1)
    nsb = nblk // G
    rows_per_tile = NP // NS
    n_arr = 2 * nch if split else nch
    mesh = plsc.VectorSubcoreMesh(core_axis_name="c", subcore_axis_name="s",
                                  num_cores=NC, num_subcores=NS)

    @functools.partial(
        pl.kernel,
        mesh=mesh,
        out_type=[jax.ShapeDtypeStruct((NP, Fc), jnp.float32)
                  for _ in range(n_arr)],
        scratch_types=[
            pltpu.VMEM((2, 2, G, EB), jnp.int32),
            pltpu.VMEM((2, EB, Fc), jnp.float32),
            pltpu.VMEM_SHARED((NP, Fc), jnp.float32),
            pltpu.SemaphoreType.DMA,
            pltpu.SemaphoreType.DMA,
            pltpu.SemaphoreType.DMA,
            pltpu.SemaphoreType.DMA,
        ],
    )
    def agg(idx_hbm, *rest):
        x_chunks = rest[:n_arr]
        out_chunks = rest[n_arr:2 * n_arr]
        idxw, bufs, acc, semi0, semi1, semg0, semg1 = rest[2 * n_arr:]
        semi = (semi0, semi1)
        semg = (semg0, semg1)
        c = lax.axis_index("c")
        s = lax.axis_index("s")
        r0 = s * rows_per_tile
        tbase = (c * (R // 2) if split else 0) + s * nblk

        def istart(g, p):
            pltpu.async_copy(idx_hbm.at[0, pl.ds(tbase + g * G, G)],
                             idxw.at[p, 0], semi[p])
            pltpu.async_copy(idx_hbm.at[1, pl.ds(tbase + g * G, G)],
                             idxw.at[p, 1], semi[p])

        def iwait(p):
            pltpu.make_async_copy(idx_hbm.at[0, pl.ds(0, G)], idxw.at[p, 0],
                                  semi[p]).wait()
            pltpu.make_async_copy(idx_hbm.at[0, pl.ds(0, G)], idxw.at[p, 1],
                                  semi[p]).wait()

        assert nsb % 2 == 1 and nsb >= 3
        for k in range(n_arr):
            @pl.when(c == (k % NC))
            def _(k=k):
                x_h = x_chunks[k]
                o_h = out_chunks[k]

                def gstart(p, j, b):
                    pltpu.async_copy(x_h.at[idxw.at[p, 0, j]],
                                     bufs.at[b], semg[b])

                def gwait(b):
                    pltpu.make_async_copy(x_h.at[idxw.at[0, 0, 0]],
                                          bufs.at[b], semg[b]).wait()

                istart(0, 0)
                istart(1, 1)
                pltpu.sync_copy(x_h.at[pl.ds(r0, rows_per_tile)],
                                acc.at[pl.ds(r0, rows_per_tile)])
                iwait(0)
                gstart(0, 0, 0)
                gstart(0, 1, 1)
                plsc.subcore_barrier()

                def sblock(g, p):
                    for j in range(G):
                        b = j % 2
                        gwait(b)
                        pltpu.sync_copy(bufs.at[b],
                                        acc.at[idxw.at[p, 1, j]],
                                        add=True)
                        if j + 2 < G:
                            gstart(p, j + 2, b)
                        else:
                            @pl.when(g + 1 < nsb)
                            def _(j=j, b=b):
                                if j + 2 == G:
                                    iwait(1 - p)
                                gstart(1 - p, j + 2 - G, b)

                    @pl.when(g + 2 < nsb)
                    def _():
                        istart(g + 2, p)

                sblock(0, 0)

                def sbpair(t, carry):
                    sblock(2 * t + 1, 1)
                    sblock(2 * t + 2, 0)
                    return carry

                lax.fori_loop(0, (nsb - 1) // 2, sbpair, 0)
                plsc.subcore_barrier()
                pltpu.sync_copy(acc.at[pl.ds(r0, rows_per_tile)],
                                o_h.at[pl.ds(r0, rows_per_tile)])
                plsc.subcore_barrier()

    return agg


@functools.cache
def _sc_deg(NP, E, eb):
    EB = eb
    R = E // EB
    nblk = R // NS // 2
    rows_per_tile = NP // NS
    mesh = plsc.VectorSubcoreMesh(core_axis_name="c", subcore_axis_name="s",
                                  num_cores=NC, num_subcores=NS)

    @functools.partial(
        pl.kernel,
        mesh=mesh,
        out_type=[jax.ShapeDtypeStruct((NP, 16), jnp.float32)
                  for _ in range(2)],
        scratch_types=[
            pltpu.VMEM((nblk, EB), jnp.int32),
            pltpu.VMEM((EB, 16), jnp.float32),
            pltpu.VMEM_SHARED((NP, 16), jnp.float32),
        ],
    )
    def deg_kernel(idx_hbm, ones_hbm, zeros_hbm, out0, out1,
                   dst_v, ones_v, acc):
        c = lax.axis_index("c")
        s = lax.axis_index("s")
        r0 = s * rows_per_tile
        pltpu.sync_copy(idx_hbm.at[1, pl.ds(c * (R // 2) + s * nblk, nblk)],
                        dst_v)

        def fill(j, carry):
            ones_v[j, :] = jnp.ones((16,), jnp.float32)
            return carry

        lax.fori_loop(0, EB, fill, 0)
        for k in range(2):
            @pl.when(c == k)
            def _(k=k):
                init_h = ones_hbm if k == 0 else zeros_hbm
                o_h = out0 if k == 0 else out1
                pltpu.sync_copy(init_h.at[pl.ds(r0, rows_per_tile)],
                                acc.at[pl.ds(r0, rows_per_tile)])
                plsc.subcore_barrier()

                def blk(j, carry):
                    pltpu.sync_copy(ones_v, acc.at[dst_v.at[j]], add=True)
                    return carry

                lax.fori_loop(0, nblk, blk, 0)
                plsc.subcore_barrier()
                pltpu.sync_copy(acc.at[pl.ds(r0, rows_per_tile)],
                                o_h.at[pl.ds(r0, rows_per_tile)])

    return deg_kernel


@functools.cache
def _t1_scale_split(NP, IN):
    nch = IN // FC

    def body(x_ref, deg0_ref, deg1_ref, dinv_ref, *outs):
        deg = deg0_ref[...][:, :1] + deg1_ref[...][:, :1]
        d = lax.rsqrt(deg)
        dinv_ref[...] = jnp.broadcast_to(d, (TB, 16))
        xs = x_ref[...] * d
        for k in range(nch):
            outs[k][...] = xs[:, k * FC:(k + 1) * FC]

    return pl.pallas_call(
        body,
        grid=(NP // TB,),
        in_specs=[pl.BlockSpec((TB, IN), lambda i: (i, 0)),
                  pl.BlockSpec((TB, 16), lambda i: (i, 0)),
                  pl.BlockSpec((TB, 16), lambda i: (i, 0))],
        out_specs=[pl.BlockSpec((TB, 16), lambda i: (i, 0))] +
                  [pl.BlockSpec((TB, FC), lambda i: (i, 0))] * nch,
        out_shape=[jax.ShapeDtypeStruct((NP, 16), jnp.float32)] +
                  [jax.ShapeDtypeStruct((NP, FC), jnp.float32)] * nch,
    )


@functools.cache
def _t2_double_matmul(NP, IN, H):
    nin = IN // FC
    nout = H // FC

    def body(*refs):
        ins = refs[:nin]
        dinv, w1t, b1, w2t = refs[nin:nin + 4]
        outs = refs[nin + 4:]
        d = dinv[...][:, :1]
        a = jnp.concatenate([r[...] for r in ins], axis=1) * d
        h1 = jnp.dot(a.astype(jnp.bfloat16), w1t[...],
                     preferred_element_type=jnp.float32)
        h1 = jnp.maximum(h1 + b1[...], 0.0)
        g = jnp.dot(h1.astype(jnp.bfloat16), w2t[...],
                    preferred_element_type=jnp.float32) * d
        for k in range(nout):
            outs[k][...] = g[:, k * FC:(k + 1) * FC]

    return pl.pallas_call(
        body,
        grid=(NP // TB,),
        in_specs=[pl.BlockSpec((TB, FC), lambda i: (i, 0))] * nin + [
            pl.BlockSpec((TB, 16), lambda i: (i, 0)),
            pl.BlockSpec((IN, H), lambda i: (0, 0)),
            pl.BlockSpec((1, H), lambda i: (0, 0)),
            pl.BlockSpec((H, H), lambda i: (0, 0)),
        ],
        out_specs=[pl.BlockSpec((TB, FC), lambda i: (i, 0))] * nout,
        out_shape=[jax.ShapeDtypeStruct((NP, FC), jnp.float32)] * nout,
    )


@functools.cache
def _t3_out_matmul(NP, N, H, CP):
    nin = H // FC

    def body(*refs):
        ins = refs[:nin]
        dinv, b2, w3t, h2_ref, g2_ref = refs[nin:]
        d = dinv[...][:, :1]
        agg = jnp.concatenate([r[...] for r in ins], axis=1)
        h2 = jnp.maximum(agg * d + b2[...], 0.0)
        h2_ref[...] = h2
        g2_ref[...] = jnp.dot(h2.astype(jnp.bfloat16), w3t[...],
                              preferred_element_type=jnp.float32) * d

    return pl.pallas_call(
        body,
        grid=(NP // TB,),
        in_specs=[pl.BlockSpec((TB, FC), lambda i: (i, 0))] * nin + [
            pl.BlockSpec((TB, 16), lambda i: (i, 0)),
            pl.BlockSpec((1, H), lambda i: (0, 0)),
            pl.BlockSpec((H, CP), lambda i: (0, 0)),
        ],
        out_specs=[pl.BlockSpec((TB, H), lambda i: (i, 0)),
                   pl.BlockSpec((TB, CP), lambda i: (i, 0))],
        out_shape=[jax.ShapeDtypeStruct((N, H), jnp.float32),
                   jax.ShapeDtypeStruct((NP, CP), jnp.float32)],
    )


@functools.cache
def _t4_final(NP, N, C, CP):
    def body(p0_ref, p1_ref, dinv_ref, b3_ref, o_ref):
        v = (p0_ref[...] + p1_ref[...]) * dinv_ref[...][:, :1]
        o_ref[...] = v[:, :C] + b3_ref[...]

    return pl.pallas_call(
        body,
        grid=(NP // TB,),
        in_specs=[pl.BlockSpec((TB, CP), lambda i: (i, 0)),
                  pl.BlockSpec((TB, CP), lambda i: (i, 0)),
                  pl.BlockSpec((TB, 16), lambda i: (i, 0)),
                  pl.BlockSpec((1, C), lambda i: (0, 0))],
        out_specs=pl.BlockSpec((TB, C), lambda i: (i, 0)),
        out_shape=jax.ShapeDtypeStruct((N, C), jnp.float32),
    )


def kernel(x, edge_index, W1, b1, W2, b2, W3, b3):
    N, IN = x.shape
    E = edge_index.shape[1]
    H = W1.shape[0]
    C = W3.shape[0]
    NP = -(-N // TB) * TB
    CP = 128

    idx3 = edge_index.reshape(2, E // EB, EB)

    ones16 = jnp.ones((NP, 16), jnp.float32)
    zeros16 = jnp.zeros((NP, 16), jnp.float32)
    zerosCP = jnp.zeros((NP, CP), jnp.float32)

    deg0, deg1 = _sc_deg(NP, E, EB)(idx3, ones16, zeros16)

    t1 = _t1_scale_split(NP, IN)
    dinv16, *xc = t1(x, deg0, deg1)

    a = _sc_agg(NP, E, IN // FC)(idx3, *xc)

    t2 = _t2_double_matmul(NP, IN, H)
    g1 = t2(*a, dinv16, W1.T.astype(jnp.bfloat16), b1[None],
            W2.T.astype(jnp.bfloat16))

    m = _sc_agg(NP, E, H // FC)(idx3, *g1)

    t3 = _t3_out_matmul(NP, N, H, CP)
    w3tp = jnp.pad(W3.T, ((0, 0), (0, CP - C))).astype(jnp.bfloat16)
    h2p, g2 = t3(*m, dinv16, b2[None], w3tp)

    p0, p1 = _sc_agg(NP, E, 1, True)(idx3, g2, zerosCP)

    t4 = _t4_final(NP, N, C, CP)
    outp = t4(p0, p1, dinv16, b3[None])

    return h2p, outp
